# baseline (speedup 1.0000x reference)
import functools

import jax
import jax.numpy as jnp
from jax import lax
from jax.experimental import pallas as pl
from jax.experimental.pallas import tpu as pltpu
from jax.experimental.pallas import tpu_sc as plsc

E = 1600000
D = 16
G = 256
H = 128

NC = 2
NS = 16
NW = NC * NS
LANES = 16

EG = E // 128
NG = 18
NCH = 21
BULK_G = NG * NCH * NW
TAIL_G = EG - BULK_G
NGT = 13
CE = NG * 128
CW = NG * 8 * 128
CWT = NGT * 8 * 128
CET = NGT * 128
HALF = (E // 128) * 8 * 128

BIG = jnp.int32(2**30)
PT = (G + 8) * D


def _sc_segment_sums(attr_flat, batch):
  mesh = plsc.VectorSubcoreMesh(
      core_axis_name="c", subcore_axis_name="s",
      num_cores=NC, num_subcores=NS)

  @functools.partial(
      pl.kernel,
      out_type=[jax.ShapeDtypeStruct((NW * PT, ), jnp.float32),
                jax.ShapeDtypeStruct((NW * PT, ), jnp.float32)],
      mesh=mesh,
      scratch_types=[
          pltpu.VMEM((2 * CW,), jnp.float32),
          pltpu.VMEM((2 * CW,), jnp.float32),
          pltpu.VMEM((2 * CW,), jnp.float32),
          pltpu.VMEM((CE,), jnp.int32),
          pltpu.VMEM((CE,), jnp.int32),
          pltpu.VMEM((CE,), jnp.int32),
          pltpu.VMEM((PT,), jnp.float32),
          pltpu.VMEM((PT,), jnp.float32),
          pltpu.SemaphoreType.DMA,
          pltpu.SemaphoreType.DMA,
          pltpu.SemaphoreType.DMA,
      ],
      compiler_params=pltpu.CompilerParams(use_tc_tiling_on_sc=False,
                                           needs_layout_passes=False),
  )
  def k(attr_hbm, batch_hbm, sums_hbm, cnt_hbm,
        attr_v0, attr_v1, attr_v2, idx_v0, idx_v1, idx_v2,
        acc_sum, acc_cnt, in_s0, in_s1, in_s2):
    cid = lax.axis_index("c")
    sid = lax.axis_index("s")
    wid = cid * NS + sid

    attr_bufs = (attr_v0, attr_v1, attr_v2)
    idx_bufs = (idx_v0, idx_v1, idx_v2)
    in_sems = (in_s0, in_s1, in_s2)

    iota = lax.iota(jnp.int32, LANES)
    lane0 = iota == 0
    zf = jnp.zeros((LANES,), jnp.float32)
    zi = jnp.zeros((LANES,), jnp.int32)

    def zbody(i, carry):
      acc_sum[pl.ds(i * LANES, LANES)] = zf
      acc_cnt[pl.ds(i * LANES, LANES)] = zf
      return carry

    lax.fori_loop(0, PT // LANES, zbody, 0)

    def start_in(j, b):
      c = wid + NW * j
      pltpu.async_copy(attr_hbm.at[pl.ds(c * CW, CW)],
                       attr_bufs[b].at[pl.ds(0, CW)], in_sems[b])
      pltpu.async_copy(attr_hbm.at[pl.ds(HALF + c * CW, CW)],
                       attr_bufs[b].at[pl.ds(CW, CW)], in_sems[b])
      pltpu.async_copy(batch_hbm.at[pl.ds(c * CE, CE)], idx_bufs[b],
                       in_sems[b])

    def wait_in(j, b):
      c = wid + NW * j
      pltpu.make_async_copy(attr_hbm.at[pl.ds(c * CW, CW)],
                            attr_bufs[b].at[pl.ds(0, CW)], in_sems[b]).wait()
      pltpu.make_async_copy(attr_hbm.at[pl.ds(HALF + c * CW, CW)],
                            attr_bufs[b].at[pl.ds(CW, CW)], in_sems[b]).wait()
      pltpu.make_async_copy(batch_hbm.at[pl.ds(c * CE, CE)], idx_bufs[b],
                            in_sems[b]).wait()

    def flush(scur, runv, cnt_v):
      srow = jnp.where(scur < 0, G, scur) * D
      for f in range(D):
        tot = jnp.full((LANES,), jnp.sum(runv[f]))
        plsc.store_scatter(acc_sum, (srow + f,), tot, mask=lane0)
      ctot = jnp.full((LANES,), jnp.sum(cnt_v))
      plsc.store_scatter(acc_cnt, (srow + iota,), ctot)

    def consume(b, ng, carry):
      attr_v = attr_bufs[b]
      idx_v = idx_bufs[b]

      def feat_vec(g, e16, f):
        off = (CW if f >= 8 else 0) + (f & 7) * 128
        return attr_v[pl.ds(g * 1024 + off + e16 * LANES, LANES)]

      def group(g, carry):
        scur, runv, cnt_v = carry[0], list(carry[1]), carry[2]
        va = idx_v[pl.ds(g * 128, LANES)]
        vb = idx_v[pl.ds(g * 128 + 112, LANES)]
        first = va[zi]
        last = vb[jnp.full((LANES,), 15, jnp.int32)]
        fast = jnp.all((first == last) & (first == scur))

        def fast_fn(carry):
          scur, runv, cnt_v = carry[0], list(carry[1]), carry[2]
          for e16 in range(8):
            for f in range(D):
              runv[f] = runv[f] + feat_vec(g, e16, f)
          return (scur, tuple(runv), cnt_v + 8.0)

        def slow_fn(carry):
          scur, runv, cnt_v = carry[0], carry[1], carry[2]

          def sub(e16, carry):
            scur, runv, cnt_v = carry[0], list(carry[1]), carry[2]
            bvec = idx_v[pl.ds(g * 128 + e16 * LANES, LANES)]
            feats = [feat_vec(g, e16, f) for f in range(D)]

            def accum(scur, runv, cnt_v):
              m = jnp.where(bvec == scur, 1.0, 0.0)
              runv = [runv[f] + feats[f] * m for f in range(D)]
              return runv, cnt_v + m

            runv, cnt_v = accum(scur, runv, cnt_v)

            def wcond(carry):
              scur = carry[0]
              return jnp.any(bvec > scur)

            def wbody(carry):
              scur, runv, cnt_v = carry[0], list(carry[1]), carry[2]
              flush(scur, runv, cnt_v)
              rem = jnp.where(bvec > scur, bvec, BIG)
              scur = jnp.full((LANES,), jnp.min(rem))
              runv, cnt_v = accum(scur, [zf] * D, zf)
              return (scur, tuple(runv), cnt_v)

            return lax.while_loop(wcond, wbody, (scur, tuple(runv), cnt_v))

          return lax.fori_loop(0, 8, sub, (scur, runv, cnt_v))

        return lax.cond(fast, fast_fn, slow_fn, (scur, tuple(runv), cnt_v))

      return lax.fori_loop(0, ng, group, carry)

    tail_s = BULK_G + (TAIL_G * wid) // NW
    tail_n = BULK_G + (TAIL_G * (wid + 1)) // NW - tail_s

    def start_tail(b):
      pltpu.async_copy(attr_hbm.at[pl.ds(tail_s * 1024, CWT)],
                       attr_bufs[b].at[pl.ds(0, CWT)], in_sems[b])
      pltpu.async_copy(attr_hbm.at[pl.ds(HALF + tail_s * 1024, CWT)],
                       attr_bufs[b].at[pl.ds(CW, CWT)], in_sems[b])
      pltpu.async_copy(batch_hbm.at[pl.ds(tail_s * 128, CET)],
                       idx_bufs[b].at[pl.ds(0, CET)], in_sems[b])

    def wait_tail(b):
      pltpu.make_async_copy(attr_hbm.at[pl.ds(tail_s * 1024, CWT)],
                            attr_bufs[b].at[pl.ds(0, CWT)],
                            in_sems[b]).wait()
      pltpu.make_async_copy(attr_hbm.at[pl.ds(HALF + tail_s * 1024, CWT)],
                            attr_bufs[b].at[pl.ds(CW, CWT)],
                            in_sems[b]).wait()
      pltpu.make_async_copy(batch_hbm.at[pl.ds(tail_s * 128, CET)],
                            idx_bufs[b].at[pl.ds(0, CET)], in_sems[b]).wait()

    carry = (jnp.full((LANES,), -1, jnp.int32), tuple([zf] * D), zf)

    start_in(0, 0)
    start_in(1, 1)
    start_in(2, 2)

    def trip(i, carry):
      for b in range(3):
        j = 3 * i + b
        wait_in(j, b)
        carry = consume(b, NG, carry)

        @pl.when(j + 3 < NCH)
        def _():
          start_in(j + 3, b)

        @pl.when(j + 3 == NCH)
        def _():
          start_tail(b)

      return carry

    carry = lax.fori_loop(0, NCH // 3, trip, carry)

    wait_tail(0)
    scur, runv, cnt_v = consume(0, tail_n, carry)
    flush(scur, list(runv), cnt_v)

    pltpu.sync_copy(acc_sum, sums_hbm.at[pl.ds(wid * PT, PT)])
    pltpu.sync_copy(acc_cnt, cnt_hbm.at[pl.ds(wid * PT, PT)])

  return k(attr_flat, batch)


def _mlp(sums, cnt, W1, b1, W2, b2):
  w1b = jnp.kron(jnp.eye(8, dtype=jnp.float32), W1)
  b1b = jnp.tile(b1, 8).reshape(1, 8 * H)

  R = PT // 128

  def body(s_ref, c_ref, w1_ref, b1_ref, w2_ref, b2_ref, out_ref):
    s = s_ref[0:R, :]
    c = c_ref[0:R, :]
    for w in range(1, NW):
      s = s + s_ref[w * R:(w + 1) * R, :]
      c = c + c_ref[w * R:(w + 1) * R, :]
    mean = s / jnp.maximum(c, 1.0)
    h = jnp.dot(mean, w1_ref[:], preferred_element_type=jnp.float32)
    h = jnp.maximum(h + b1_ref[:], 0.0).reshape(PT // D, H)
    out = jnp.dot(h, w2_ref[:], preferred_element_type=jnp.float32)
    out_ref[:] = out[:G, :] + b2_ref[:]

  return pl.pallas_call(
      body,
      out_shape=jax.ShapeDtypeStruct((G, H), jnp.float32),
  )(sums.reshape(NW * PT // 128, 128), cnt.reshape(NW * PT // 128, 128),
    w1b, b1b, W2, b2.reshape(1, H))


def kernel(edge_attr, edge_batch, W1, b1, W2, b2):
  attr_flat = (edge_attr.reshape(EG, 128, 2, 8)
               .transpose(2, 0, 3, 1)
               .reshape(E * D))
  batch = edge_batch.astype(jnp.int32)
  sums, cnt = _sc_segment_sums(attr_flat, batch)
  return _mlp(sums, cnt, W1, b1, W2, b2)


# speedup vs baseline: 1.0221x; 1.0221x over previous
import functools

import jax
import jax.numpy as jnp
from jax import lax
from jax.experimental import pallas as pl
from jax.experimental.pallas import tpu as pltpu
from jax.experimental.pallas import tpu_sc as plsc

E = 1600000
D = 16
G = 256
H = 128

NC = 2
NS = 16
NW = NC * NS
LANES = 16

EG = E // 128
NG = 20
NCH = 19
BULK_G = NG * NCH * NW
TAIL_G = EG - BULK_G
NGT = 11
CE = NG * 128
CW = NG * 8 * 128
CWT = NGT * 8 * 128
CET = NGT * 128
HALF = (E // 128) * 8 * 128

BIG = jnp.int32(2**30)
PT = (G + 8) * D


def _sc_segment_sums(attr_flat, batch):
  mesh = plsc.VectorSubcoreMesh(
      core_axis_name="c", subcore_axis_name="s",
      num_cores=NC, num_subcores=NS)

  @functools.partial(
      pl.kernel,
      out_type=[jax.ShapeDtypeStruct((NW * PT, ), jnp.float32),
                jax.ShapeDtypeStruct((NW * PT, ), jnp.float32)],
      mesh=mesh,
      scratch_types=[
          pltpu.VMEM((2 * CW,), jnp.float32),
          pltpu.VMEM((2 * CW,), jnp.float32),
          pltpu.VMEM((CE,), jnp.int32),
          pltpu.VMEM((CE,), jnp.int32),
          pltpu.VMEM((PT,), jnp.float32),
          pltpu.VMEM((PT,), jnp.float32),
          pltpu.SemaphoreType.DMA,
          pltpu.SemaphoreType.DMA,
      ],
      compiler_params=pltpu.CompilerParams(use_tc_tiling_on_sc=False,
                                           needs_layout_passes=False),
  )
  def k(attr_hbm, batch_hbm, sums_hbm, cnt_hbm,
        attr_v0, attr_v1, idx_v0, idx_v1, acc_sum, acc_cnt, in_s0, in_s1):
    cid = lax.axis_index("c")
    sid = lax.axis_index("s")
    wid = cid * NS + sid

    attr_bufs = (attr_v0, attr_v1)
    idx_bufs = (idx_v0, idx_v1)
    in_sems = (in_s0, in_s1)

    iota = lax.iota(jnp.int32, LANES)
    lane0 = iota == 0
    zf = jnp.zeros((LANES,), jnp.float32)
    zi = jnp.zeros((LANES,), jnp.int32)

    def zbody(i, carry):
      acc_sum[pl.ds(i * LANES, LANES)] = zf
      acc_cnt[pl.ds(i * LANES, LANES)] = zf
      return carry

    lax.fori_loop(0, PT // LANES, zbody, 0)

    def start_in(j, b):
      c = wid + NW * j
      pltpu.async_copy(attr_hbm.at[pl.ds(c * CW, CW)],
                       attr_bufs[b].at[pl.ds(0, CW)], in_sems[b])
      pltpu.async_copy(attr_hbm.at[pl.ds(HALF + c * CW, CW)],
                       attr_bufs[b].at[pl.ds(CW, CW)], in_sems[b])
      pltpu.async_copy(batch_hbm.at[pl.ds(c * CE, CE)], idx_bufs[b],
                       in_sems[b])

    def wait_in(j, b):
      c = wid + NW * j
      pltpu.make_async_copy(attr_hbm.at[pl.ds(c * CW, CW)],
                            attr_bufs[b].at[pl.ds(0, CW)], in_sems[b]).wait()
      pltpu.make_async_copy(attr_hbm.at[pl.ds(HALF + c * CW, CW)],
                            attr_bufs[b].at[pl.ds(CW, CW)], in_sems[b]).wait()
      pltpu.make_async_copy(batch_hbm.at[pl.ds(c * CE, CE)], idx_bufs[b],
                            in_sems[b]).wait()

    def flush(scur, runv, cnt_v):
      srow = jnp.where(scur < 0, G, scur) * D
      for f in range(D):
        tot = jnp.full((LANES,), jnp.sum(runv[f]))
        plsc.store_scatter(acc_sum, (srow + f,), tot, mask=lane0)
      ctot = jnp.full((LANES,), jnp.sum(cnt_v))
      plsc.store_scatter(acc_cnt, (srow + iota,), ctot)

    def consume(b, ng, carry):
      attr_v = attr_bufs[b]
      idx_v = idx_bufs[b]

      def feat_vec(g, e16, f):
        off = (CW if f >= 8 else 0) + (f & 7) * 128
        return attr_v[pl.ds(g * 1024 + off + e16 * LANES, LANES)]

      def group(g, carry):
        scur, runv, cnt_v = carry[0], list(carry[1]), carry[2]
        va = idx_v[pl.ds(g * 128, LANES)]
        vb = idx_v[pl.ds(g * 128 + 112, LANES)]
        first = va[zi]
        last = vb[jnp.full((LANES,), 15, jnp.int32)]
        fast = jnp.all((first == last) & (first == scur))

        def fast_fn(carry):
          scur, runv, cnt_v = carry[0], list(carry[1]), carry[2]
          for e16 in range(8):
            for f in range(D):
              runv[f] = runv[f] + feat_vec(g, e16, f)
          return (scur, tuple(runv), cnt_v + 8.0)

        def slow_fn(carry):
          scur, runv, cnt_v = carry[0], carry[1], carry[2]

          def sub(e16, carry):
            scur, runv, cnt_v = carry[0], list(carry[1]), carry[2]
            bvec = idx_v[pl.ds(g * 128 + e16 * LANES, LANES)]
            feats = [feat_vec(g, e16, f) for f in range(D)]

            def accum(scur, runv, cnt_v):
              m = jnp.where(bvec == scur, 1.0, 0.0)
              runv = [runv[f] + feats[f] * m for f in range(D)]
              return runv, cnt_v + m

            runv, cnt_v = accum(scur, runv, cnt_v)

            def wcond(carry):
              scur = carry[0]
              return jnp.any(bvec > scur)

            def wbody(carry):
              scur, runv, cnt_v = carry[0], list(carry[1]), carry[2]
              flush(scur, runv, cnt_v)
              rem = jnp.where(bvec > scur, bvec, BIG)
              scur = jnp.full((LANES,), jnp.min(rem))
              runv, cnt_v = accum(scur, [zf] * D, zf)
              return (scur, tuple(runv), cnt_v)

            return lax.while_loop(wcond, wbody, (scur, tuple(runv), cnt_v))

          return lax.fori_loop(0, 8, sub, (scur, runv, cnt_v))

        return lax.cond(fast, fast_fn, slow_fn, (scur, tuple(runv), cnt_v))

      return lax.fori_loop(0, ng, group, carry)

    tail_s = BULK_G + (TAIL_G * wid) // NW
    tail_n = BULK_G + (TAIL_G * (wid + 1)) // NW - tail_s

    def start_tail(b):
      pltpu.async_copy(attr_hbm.at[pl.ds(tail_s * 1024, CWT)],
                       attr_bufs[b].at[pl.ds(0, CWT)], in_sems[b])
      pltpu.async_copy(attr_hbm.at[pl.ds(HALF + tail_s * 1024, CWT)],
                       attr_bufs[b].at[pl.ds(CW, CWT)], in_sems[b])
      pltpu.async_copy(batch_hbm.at[pl.ds(tail_s * 128, CET)],
                       idx_bufs[b].at[pl.ds(0, CET)], in_sems[b])

    def wait_tail(b):
      pltpu.make_async_copy(attr_hbm.at[pl.ds(tail_s * 1024, CWT)],
                            attr_bufs[b].at[pl.ds(0, CWT)],
                            in_sems[b]).wait()
      pltpu.make_async_copy(attr_hbm.at[pl.ds(HALF + tail_s * 1024, CWT)],
                            attr_bufs[b].at[pl.ds(CW, CWT)],
                            in_sems[b]).wait()
      pltpu.make_async_copy(batch_hbm.at[pl.ds(tail_s * 128, CET)],
                            idx_bufs[b].at[pl.ds(0, CET)], in_sems[b]).wait()

    carry = (jnp.full((LANES,), -1, jnp.int32), tuple([zf] * D), zf)

    start_in(0, 0)
    start_in(1, 1)

    def pair(i, carry):
      j0 = 2 * i
      j1 = j0 + 1
      wait_in(j0, 0)
      carry = consume(0, NG, carry)

      @pl.when(j0 + 2 < NCH)
      def _():
        start_in(j0 + 2, 0)

      wait_in(j1, 1)
      carry = consume(1, NG, carry)

      @pl.when(j1 + 2 < NCH)
      def _():
        start_in(j1 + 2, 1)

      return carry

    carry = lax.fori_loop(0, NCH // 2, pair, carry)

    wait_in(NCH - 1, 0)
    start_tail(1)
    carry = consume(0, NG, carry)
    wait_tail(1)
    scur, runv, cnt_v = consume(1, tail_n, carry)
    flush(scur, list(runv), cnt_v)

    pltpu.sync_copy(acc_sum, sums_hbm.at[pl.ds(wid * PT, PT)])
    pltpu.sync_copy(acc_cnt, cnt_hbm.at[pl.ds(wid * PT, PT)])

  return k(attr_flat, batch)


def _mlp(sums, cnt, W1, b1, W2, b2):
  w1b = jnp.kron(jnp.eye(8, dtype=jnp.float32), W1)
  b1b = jnp.tile(b1, 8).reshape(1, 8 * H)

  R = PT // 128

  def body(s_ref, c_ref, w1_ref, b1_ref, w2_ref, b2_ref, out_ref):
    s = s_ref[0:R, :]
    c = c_ref[0:R, :]
    for w in range(1, NW):
      s = s + s_ref[w * R:(w + 1) * R, :]
      c = c + c_ref[w * R:(w + 1) * R, :]
    mean = s / jnp.maximum(c, 1.0)
    h = jnp.dot(mean, w1_ref[:], preferred_element_type=jnp.float32)
    h = jnp.maximum(h + b1_ref[:], 0.0).reshape(PT // D, H)
    out = jnp.dot(h, w2_ref[:], preferred_element_type=jnp.float32)
    out_ref[:] = out[:G, :] + b2_ref[:]

  return pl.pallas_call(
      body,
      out_shape=jax.ShapeDtypeStruct((G, H), jnp.float32),
  )(sums.reshape(NW * PT // 128, 128), cnt.reshape(NW * PT // 128, 128),
    w1b, b1b, W2, b2.reshape(1, H))


def kernel(edge_attr, edge_batch, W1, b1, W2, b2):
  attr_flat = (edge_attr.reshape(EG, 128, 2, 8)
               .transpose(2, 0, 3, 1)
               .reshape(E * D))
  batch = edge_batch.astype(jnp.int32)
  sums, cnt = _sc_segment_sums(attr_flat, batch)
  return _mlp(sums, cnt, W1, b1, W2, b2)
